# baseline (speedup 1.0000x reference)
import functools

import jax
import jax.numpy as jnp
from jax.experimental import pallas as pl
from jax.experimental.pallas import tpu as pltpu

T = 2048
D = 1024
F = 2048
E = 8
B = 128
G = T // B + E


def _router_kernel(x_ref, wr_ref, idx_ref, val_ref, aux_ref):
    x = x_ref[...]
    wr = wr_ref[...]
    logits = jnp.dot(x, wr, preferred_element_type=jnp.float32)
    m = jnp.max(logits, axis=-1, keepdims=True)
    ex = jnp.exp(logits - m)
    s = jnp.sum(ex, axis=-1, keepdims=True)
    gates = ex / s
    iota = jax.lax.broadcasted_iota(jnp.int32, logits.shape, 1)
    idx = jnp.min(jnp.where(logits >= m, iota, E), axis=-1)
    one_hot = (iota == idx[:, None]).astype(jnp.float32)
    importance = jnp.sum(gates, axis=0)
    load = jnp.sum(one_hot, axis=0)
    aux = (E / (T * T)) * jnp.sum(importance * load)
    idx_ref[...] = idx[:, None]
    val_ref[...] = 1.0 / s
    aux_ref[...] = jnp.reshape(aux, (1, 1))


def _ffn_kernel(perm_ref, be_ref, bstart_ref, bvalid_ref,
                rfirst_ref, fetch_ref, ne_ref, slot_ref,
                x_ref, w1_any, w2_any, val_ref,
                out_ref,
                w1b_ref, w2b_ref, sem1, sem2,
                xb_ref, vb_ref, yb_ref):
    g = pl.program_id(0)
    start = bstart_ref[g]
    valid = bvalid_ref[g]
    slot = slot_ref[g]

    @pl.when(g == 0)
    def _():
        e0 = be_ref[0]
        pltpu.make_async_copy(w1_any.at[e0], w1b_ref.at[0], sem1.at[0]).start()
        pltpu.make_async_copy(w2_any.at[e0], w2b_ref.at[0], sem2.at[0]).start()

    @pl.when(fetch_ref[g] == 1)
    def _():
        ne = ne_ref[g]
        ns = 1 - slot
        pltpu.make_async_copy(w1_any.at[ne], w1b_ref.at[ns], sem1.at[ns]).start()
        pltpu.make_async_copy(w2_any.at[ne], w2b_ref.at[ns], sem2.at[ns]).start()

    @pl.when(valid > 0)
    def _():
        nfull = (valid // 8) * 8

        def gather8(c, _):
            b0 = c * 8
            for u in range(8):
                i = b0 + u
                tok = perm_ref[start + i]
                xb_ref[i, :] = x_ref[tok, :]
                vb_ref[i, :] = val_ref[tok, :]
            return 0
        jax.lax.fori_loop(0, valid // 8, gather8, 0)

        def gather1(i, _):
            tok = perm_ref[start + i]
            xb_ref[i, :] = x_ref[tok, :]
            vb_ref[i, :] = val_ref[tok, :]
            return 0
        jax.lax.fori_loop(nfull, valid, gather1, 0)

    @pl.when(rfirst_ref[g] == 1)
    def _():
        e = be_ref[g]
        pltpu.make_async_copy(w1_any.at[e], w1b_ref.at[slot], sem1.at[slot]).wait()
        pltpu.make_async_copy(w2_any.at[e], w2b_ref.at[slot], sem2.at[slot]).wait()

    @pl.when(valid > B // 2)
    def _():
        h = jnp.maximum(
            jnp.dot(xb_ref[...], w1b_ref[slot],
                    preferred_element_type=jnp.float32),
            0.0)
        yb_ref[...] = jnp.dot(h, w2b_ref[slot],
                              preferred_element_type=jnp.float32) * vb_ref[...]

    @pl.when((valid > 0) & (valid <= B // 2))
    def _():
        xh = xb_ref[pl.ds(0, B // 2), :]
        h = jnp.maximum(
            jnp.dot(xh, w1b_ref[slot], preferred_element_type=jnp.float32),
            0.0)
        yb_ref[pl.ds(0, B // 2), :] = jnp.dot(
            h, w2b_ref[slot],
            preferred_element_type=jnp.float32) * vb_ref[pl.ds(0, B // 2), :]

    @pl.when(valid > 0)
    def _():
        nfull = (valid // 8) * 8

        def scatter8(c, _):
            b0 = c * 8
            for u in range(8):
                i = b0 + u
                tok = perm_ref[start + i]
                out_ref[tok, :] = yb_ref[i, :]
            return 0
        jax.lax.fori_loop(0, valid // 8, scatter8, 0)

        def scatter1(i, _):
            tok = perm_ref[start + i]
            out_ref[tok, :] = yb_ref[i, :]
            return 0
        jax.lax.fori_loop(nfull, valid, scatter1, 0)


@jax.jit
def kernel(x, Wr, W1, W2):
    idx2, val2, aux2 = pl.pallas_call(
        _router_kernel,
        out_shape=(
            jax.ShapeDtypeStruct((T, 1), jnp.int32),
            jax.ShapeDtypeStruct((T, 1), jnp.float32),
            jax.ShapeDtypeStruct((1, 1), jnp.float32),
        ),
    )(x, Wr)
    top_idx = idx2[:, 0]

    perm = jnp.argsort(top_idx, stable=True).astype(jnp.int32)
    counts = jnp.sum((top_idx[:, None] == jnp.arange(E)[None, :]).astype(jnp.int32),
                     axis=0)
    offsets = jnp.concatenate([jnp.zeros((1,), jnp.int32),
                               jnp.cumsum(counts)[:-1].astype(jnp.int32)])
    nblk = (counts + B - 1) // B
    blk_cum = jnp.concatenate([jnp.zeros((1,), jnp.int32),
                               jnp.cumsum(nblk)[:-1].astype(jnp.int32)])
    gid = jnp.arange(G, dtype=jnp.int32)
    be = jnp.sum((blk_cum[None, :] <= gid[:, None]).astype(jnp.int32), axis=1) - 1
    k = gid - blk_cum[be]
    bstart = offsets[be] + k * B
    bvalid = jnp.clip(counts[be] - k * B, 0, B)

    rfirst = jnp.concatenate([jnp.ones((1,), jnp.int32),
                              (be[1:] != be[:-1]).astype(jnp.int32)])
    run_idx = jnp.cumsum(rfirst) - 1
    slot = (run_idx % 2).astype(jnp.int32)
    later = gid[None, :] > gid[:, None]
    differs = be[None, :] != be[:, None]
    nxt_change = jnp.min(jnp.where(later & differs, gid[None, :], G - 1),
                         axis=1)
    ne = be[nxt_change]
    fetch = rfirst * (ne != be).astype(jnp.int32)

    grid_spec = pltpu.PrefetchScalarGridSpec(
        num_scalar_prefetch=8,
        grid=(G,),
        in_specs=[
            pl.BlockSpec((T, D), lambda g, *_: (0, 0)),
            pl.BlockSpec(memory_space=pltpu.MemorySpace.HBM),
            pl.BlockSpec(memory_space=pltpu.MemorySpace.HBM),
            pl.BlockSpec((T, 1), lambda g, *_: (0, 0)),
        ],
        out_specs=pl.BlockSpec((T, D), lambda g, *_: (0, 0)),
        scratch_shapes=[
            pltpu.VMEM((2, D, F), jnp.float32),
            pltpu.VMEM((2, F, D), jnp.float32),
            pltpu.SemaphoreType.DMA((2,)),
            pltpu.SemaphoreType.DMA((2,)),
            pltpu.VMEM((B, D), jnp.float32),
            pltpu.VMEM((B, 1), jnp.float32),
            pltpu.VMEM((B, D), jnp.float32),
        ],
    )
    out = pl.pallas_call(
        _ffn_kernel,
        grid_spec=grid_spec,
        out_shape=jax.ShapeDtypeStruct((T, D), jnp.float32),
        compiler_params=pltpu.CompilerParams(
            dimension_semantics=("arbitrary",)),
    )(perm, be, bstart, bvalid, rfirst, fetch, ne, slot, x, W1, W2, val2)

    return out, top_idx, aux2[0, 0]


# speedup vs baseline: 1.0213x; 1.0213x over previous
import functools

import jax
import jax.numpy as jnp
from jax.experimental import pallas as pl
from jax.experimental.pallas import tpu as pltpu

T = 2048
D = 1024
F = 2048
E = 8
B = 128
G = T // B + E


def _router_kernel(x_ref, wr_ref, idx_ref, val_ref, aux_ref):
    x = x_ref[...]
    wr = wr_ref[...]
    logits = jnp.dot(x, wr, preferred_element_type=jnp.float32)
    m = jnp.max(logits, axis=-1, keepdims=True)
    ex = jnp.exp(logits - m)
    s = jnp.sum(ex, axis=-1, keepdims=True)
    gates = ex / s
    iota = jax.lax.broadcasted_iota(jnp.int32, logits.shape, 1)
    idx = jnp.min(jnp.where(logits >= m, iota, E), axis=-1)
    one_hot = (iota == idx[:, None]).astype(jnp.float32)
    importance = jnp.sum(gates, axis=0)
    load = jnp.sum(one_hot, axis=0)
    aux = (E / (T * T)) * jnp.sum(importance * load)
    idx_ref[...] = idx[:, None]
    val_ref[...] = 1.0 / s
    aux_ref[...] = jnp.reshape(aux, (1, 1))


def _ffn_kernel(perm_ref, be_ref, bstart_ref, bvalid_ref,
                rfirst_ref, fetch_ref, ne_ref, slot_ref,
                x_ref, w1_any, w2_any, val_ref,
                out_ref,
                w1b_ref, w2b_ref, sem1, sem2,
                xb_ref, vb_ref, yb_ref):
    g = pl.program_id(0)
    start = bstart_ref[g]
    valid = bvalid_ref[g]
    slot = slot_ref[g]

    @pl.when(g == 0)
    def _():
        e0 = be_ref[0]
        pltpu.make_async_copy(w1_any.at[e0], w1b_ref.at[0], sem1.at[0]).start()
        pltpu.make_async_copy(w2_any.at[e0], w2b_ref.at[0], sem2.at[0]).start()

    @pl.when(fetch_ref[g] == 1)
    def _():
        ne = ne_ref[g]
        ns = 1 - slot
        pltpu.make_async_copy(w1_any.at[ne], w1b_ref.at[ns], sem1.at[ns]).start()
        pltpu.make_async_copy(w2_any.at[ne], w2b_ref.at[ns], sem2.at[ns]).start()

    @pl.when(valid > 0)
    def _():
        nfull = (valid // 8) * 8

        def gather8(c, _):
            b0 = c * 8
            for u in range(8):
                i = b0 + u
                tok = perm_ref[start + i]
                xb_ref[i, :] = x_ref[tok, :]
                vb_ref[i, :] = val_ref[tok, :]
            return 0
        jax.lax.fori_loop(0, valid // 8, gather8, 0)

        def gather1(i, _):
            tok = perm_ref[start + i]
            xb_ref[i, :] = x_ref[tok, :]
            vb_ref[i, :] = val_ref[tok, :]
            return 0
        jax.lax.fori_loop(nfull, valid, gather1, 0)

    @pl.when(rfirst_ref[g] == 1)
    def _():
        e = be_ref[g]
        pltpu.make_async_copy(w1_any.at[e], w1b_ref.at[slot], sem1.at[slot]).wait()
        pltpu.make_async_copy(w2_any.at[e], w2b_ref.at[slot], sem2.at[slot]).wait()

    @pl.when(valid > 0)
    def _():
        h = jnp.maximum(
            jnp.dot(xb_ref[...], w1b_ref[slot],
                    preferred_element_type=jnp.float32),
            0.0)
        yb_ref[...] = jnp.dot(h, w2b_ref[slot],
                              preferred_element_type=jnp.float32) * vb_ref[...]

    @pl.when(valid > 0)
    def _():
        nfull = (valid // 8) * 8

        def scatter8(c, _):
            b0 = c * 8
            for u in range(8):
                i = b0 + u
                tok = perm_ref[start + i]
                out_ref[tok, :] = yb_ref[i, :]
            return 0
        jax.lax.fori_loop(0, valid // 8, scatter8, 0)

        def scatter1(i, _):
            tok = perm_ref[start + i]
            out_ref[tok, :] = yb_ref[i, :]
            return 0
        jax.lax.fori_loop(nfull, valid, scatter1, 0)


@jax.jit
def kernel(x, Wr, W1, W2):
    idx2, val2, aux2 = pl.pallas_call(
        _router_kernel,
        out_shape=(
            jax.ShapeDtypeStruct((T, 1), jnp.int32),
            jax.ShapeDtypeStruct((T, 1), jnp.float32),
            jax.ShapeDtypeStruct((1, 1), jnp.float32),
        ),
    )(x, Wr)
    top_idx = idx2[:, 0]

    perm = jnp.argsort(top_idx, stable=True).astype(jnp.int32)
    counts = jnp.sum((top_idx[:, None] == jnp.arange(E)[None, :]).astype(jnp.int32),
                     axis=0)
    offsets = jnp.concatenate([jnp.zeros((1,), jnp.int32),
                               jnp.cumsum(counts)[:-1].astype(jnp.int32)])
    nblk = (counts + B - 1) // B
    blk_cum = jnp.concatenate([jnp.zeros((1,), jnp.int32),
                               jnp.cumsum(nblk)[:-1].astype(jnp.int32)])
    gid = jnp.arange(G, dtype=jnp.int32)
    be = jnp.sum((blk_cum[None, :] <= gid[:, None]).astype(jnp.int32), axis=1) - 1
    k = gid - blk_cum[be]
    bstart = offsets[be] + k * B
    bvalid = jnp.clip(counts[be] - k * B, 0, B)

    rfirst = jnp.concatenate([jnp.ones((1,), jnp.int32),
                              (be[1:] != be[:-1]).astype(jnp.int32)])
    run_idx = jnp.cumsum(rfirst) - 1
    slot = (run_idx % 2).astype(jnp.int32)
    later = gid[None, :] > gid[:, None]
    differs = be[None, :] != be[:, None]
    nxt_change = jnp.min(jnp.where(later & differs, gid[None, :], G - 1),
                         axis=1)
    ne = be[nxt_change]
    fetch = rfirst * (ne != be).astype(jnp.int32)

    grid_spec = pltpu.PrefetchScalarGridSpec(
        num_scalar_prefetch=8,
        grid=(G,),
        in_specs=[
            pl.BlockSpec((T, D), lambda g, *_: (0, 0)),
            pl.BlockSpec(memory_space=pltpu.MemorySpace.HBM),
            pl.BlockSpec(memory_space=pltpu.MemorySpace.HBM),
            pl.BlockSpec((T, 1), lambda g, *_: (0, 0)),
        ],
        out_specs=pl.BlockSpec((T, D), lambda g, *_: (0, 0)),
        scratch_shapes=[
            pltpu.VMEM((2, D, F), jnp.float32),
            pltpu.VMEM((2, F, D), jnp.float32),
            pltpu.SemaphoreType.DMA((2,)),
            pltpu.SemaphoreType.DMA((2,)),
            pltpu.VMEM((B, D), jnp.float32),
            pltpu.VMEM((B, 1), jnp.float32),
            pltpu.VMEM((B, D), jnp.float32),
        ],
    )
    out = pl.pallas_call(
        _ffn_kernel,
        grid_spec=grid_spec,
        out_shape=jax.ShapeDtypeStruct((T, D), jnp.float32),
        compiler_params=pltpu.CompilerParams(
            dimension_semantics=("arbitrary",)),
    )(perm, be, bstart, bvalid, rfirst, fetch, ne, slot, x, W1, W2, val2)

    return out, top_idx, aux2[0, 0]
